# Initial kernel scaffold; baseline (speedup 1.0000x reference)
import jax
import jax.numpy as jnp
from jax.experimental import pallas as pl


def kernel(z_1, z_2, dist_grade):
    raise NotImplementedError("write your pallas kernel here")


# speedup vs baseline: 4.1326x; 4.1326x over previous
import functools

import jax
import jax.numpy as jnp
from jax import lax
from jax.experimental import pallas as pl
from jax.experimental.pallas import tpu as pltpu
from jax.experimental.pallas import tpu_sc as plsc


@functools.lru_cache(maxsize=None)
def _make_sc_call(B, D, G):
    info = plsc.get_sparse_core_info()
    NC, NS, L = info.num_cores, info.num_subcores, info.num_lanes
    NW = NC * NS
    BW = B // NW
    CR = 1024
    NCHUNK = BW // CR
    GROUPS = CR // L
    assert B % (NW * CR) == 0 and CR % L == 0

    thr = [4.0 * j * j / (G * G) for j in range(1, G)]

    mesh = plsc.VectorSubcoreMesh(core_axis_name="c", subcore_axis_name="s")

    @functools.partial(
        pl.kernel,
        mesh=mesh,
        out_type=jax.ShapeDtypeStruct((B,), jnp.float32),
        compiler_params=pltpu.CompilerParams(needs_layout_passes=False),
        scratch_types=[
            pltpu.VMEM((CR * D,), jnp.float32),
            pltpu.VMEM((CR * D,), jnp.float32),
            pltpu.VMEM((CR * G,), jnp.float32),
            pltpu.VMEM((CR,), jnp.float32),
        ],
    )
    def sc_kernel(z1_hbm, z2_hbm, dg_hbm, out_hbm, z1v, z2v, dgv, outv):
        wid = lax.axis_index("s") * NC + lax.axis_index("c")
        row0 = wid * BW
        lanes = lax.iota(jnp.int32, L)
        izd = lanes * D
        izg = lanes * G

        def chunk_body(ci, carry):
            base = row0 + ci * CR
            pltpu.sync_copy(z1_hbm.at[pl.ds(base * D, CR * D)], z1v)
            pltpu.sync_copy(z2_hbm.at[pl.ds(base * D, CR * D)], z2v)
            pltpu.sync_copy(dg_hbm.at[pl.ds(base * G, CR * G)], dgv)

            def group_body(g, c2):
                zb = izd + g * (L * D)
                a0 = plsc.load_gather(z1v, [zb])
                a1 = plsc.load_gather(z1v, [zb + 1])
                a2 = plsc.load_gather(z1v, [zb + 2])
                a3 = plsc.load_gather(z1v, [zb + 3])
                b0 = plsc.load_gather(z2v, [zb])
                b1 = plsc.load_gather(z2v, [zb + 1])
                b2 = plsc.load_gather(z2v, [zb + 2])
                b3 = plsc.load_gather(z2v, [zb + 3])
                dx = (b0 + b2) - (a0 + a2)
                dy = (b1 + b3) - (a1 + a3)
                r2 = dx * dx + dy * dy
                did = (r2 >= thr[0]).astype(jnp.int32)
                for t in thr[1:]:
                    did = did + (r2 >= t).astype(jnp.int32)
                gi = izg + (g * (L * G) + did)
                outv[pl.ds(g * L, L)] = plsc.load_gather(dgv, [gi])
                return c2

            lax.fori_loop(0, GROUPS, group_body, 0, unroll=4)
            pltpu.sync_copy(outv, out_hbm.at[pl.ds(base, CR)])
            return carry

        lax.fori_loop(0, NCHUNK, chunk_body, 0)

    return sc_kernel


def kernel(z_1, z_2, dist_grade):
    B, D = z_1.shape
    G = dist_grade.shape[1]
    call = _make_sc_call(B, D, G)
    return call(z_1.reshape(-1), z_2.reshape(-1), dist_grade.reshape(-1))
